# Initial kernel scaffold; baseline (speedup 1.0000x reference)
import jax
import jax.numpy as jnp
from jax.experimental import pallas as pl


def kernel(original_kpts, segment):
    raise NotImplementedError("write your pallas kernel here")


# speedup vs baseline: 1.1660x; 1.1660x over previous
import functools

import jax
import jax.numpy as jnp
from jax import lax
from jax.experimental import pallas as pl
from jax.experimental.pallas import tpu as pltpu
from jax.experimental.pallas import tpu_sc as plsc

B, M, C, H, W = 4, 4096, 96, 384, 384
HW = H * W
NW = 32
MPW = B * M // NW


def _sc_gather(kpts_flat, seg_flat):
    mesh = plsc.VectorSubcoreMesh(core_axis_name="c", subcore_axis_name="s")

    @functools.partial(
        pl.kernel,
        mesh=mesh,
        out_type=jax.ShapeDtypeStruct((B * M * C,), jnp.float32),
        scratch_types=[
            pltpu.VMEM((2 * MPW,), jnp.float32),
            pltpu.VMEM((MPW * C,), jnp.int32),
            pltpu.VMEM((MPW * C,), jnp.float32),
            pltpu.SemaphoreType.DMA,
        ],
        compiler_params=pltpu.CompilerParams(needs_layout_passes=False),
    )
    def k(kpts_hbm, seg_hbm, out_hbm, kptv, idxv, datav, sem):
        wid = lax.axis_index("s") * 2 + lax.axis_index("c")
        row0 = wid * MPW
        b = row0 // M
        pltpu.sync_copy(kpts_hbm.at[pl.ds(row0 * 2, 2 * MPW)], kptv)

        iota = lax.iota(jnp.int32, 16)
        cbase = b * (C * HW)

        def body(g, carry):
            k0 = g * 16
            ix = (k0 + iota) * 2
            xf = plsc.load_gather(kptv, [ix])
            yf = plsc.load_gather(kptv, [ix + 1])
            xi = (xf * float(W)).astype(jnp.int32)
            yi = (yf * float(H)).astype(jnp.int32)
            xi = jnp.minimum(jnp.maximum(xi, 0), H - 1)
            yi = jnp.minimum(jnp.maximum(yi, 0), W - 1)
            pix = xi * W + yi + cbase
            pos = (k0 + iota) * C
            for c in range(C):
                plsc.store_scatter(idxv, [pos + c], pix + c * HW)
            return carry

        lax.fori_loop(0, MPW // 16, body, 0)

        pltpu.async_copy(seg_hbm.at[idxv], datav, sem).wait()
        pltpu.sync_copy(datav, out_hbm.at[pl.ds(row0 * C, MPW * C)])

    return k(kpts_flat, seg_flat)


def kernel(original_kpts, segment):
    kpts_flat = original_kpts.reshape(-1)
    seg_flat = segment.reshape(-1)
    out = _sc_gather(kpts_flat, seg_flat)
    return out.reshape(B, M, C)


# speedup vs baseline: 3.1087x; 2.6661x over previous
import functools

import jax
import jax.numpy as jnp
from jax import lax
from jax.experimental import pallas as pl
from jax.experimental.pallas import tpu as pltpu
from jax.experimental.pallas import tpu_sc as plsc

B, M, C, H, W = 4, 4096, 96, 384, 384
HW = H * W
NW = 32
MPW = B * M // NW


def _sc_gather(kpts_flat, seg_flat):
    mesh = plsc.VectorSubcoreMesh(core_axis_name="c", subcore_axis_name="s")

    @functools.partial(
        pl.kernel,
        mesh=mesh,
        out_type=jax.ShapeDtypeStruct((B * M * C,), jnp.float32),
        scratch_types=[
            pltpu.VMEM((2 * MPW,), jnp.float32),
            pltpu.VMEM((MPW * C,), jnp.int32),
            pltpu.VMEM((MPW * C,), jnp.float32),
            pltpu.SemaphoreType.DMA,
        ],
        compiler_params=pltpu.CompilerParams(needs_layout_passes=False),
    )
    def k(kpts_hbm, seg_hbm, out_hbm, kptv, idxv, datav, sem):
        wid = lax.axis_index("s") * 2 + lax.axis_index("c")
        row0 = wid * MPW
        b = row0 // M
        pltpu.sync_copy(kpts_hbm.at[pl.ds(row0 * 2, 2 * MPW)], kptv)

        iota = lax.iota(jnp.int32, 16)
        cbase = b * (C * HW)

        def body(g, carry):
            k0 = g * 16
            ix = (k0 + iota) * 2
            xf = plsc.load_gather(kptv, [ix])
            yf = plsc.load_gather(kptv, [ix + 1])
            xi = (xf * float(W)).astype(jnp.int32)
            yi = (yf * float(H)).astype(jnp.int32)
            xi = jnp.minimum(jnp.maximum(xi, 0), H - 1)
            yi = jnp.minimum(jnp.maximum(yi, 0), W - 1)
            pix = (
                (xi >> 3) * 3072
                + (yi >> 7) * 1024
                + (xi & 7) * 128
                + (yi & 127)
                + cbase
            )
            pos = (k0 + iota) * C
            for c in range(C):
                plsc.store_scatter(idxv, [pos + c], pix + c * HW)
            return carry

        lax.fori_loop(0, MPW // 16, body, 0)

        pltpu.async_copy(seg_hbm.at[idxv], datav, sem).wait()
        pltpu.sync_copy(datav, out_hbm.at[pl.ds(row0 * C, MPW * C)])

    return k(kpts_flat, seg_flat)


def kernel(original_kpts, segment):
    kpts_flat = original_kpts.reshape(-1)
    seg_phys = (
        segment.reshape(B, C, H // 8, 8, W // 128, 128)
        .transpose(0, 1, 2, 4, 3, 5)
        .reshape(-1)
    )
    out = _sc_gather(kpts_flat, seg_phys)
    return out.reshape(B, M, C)


# speedup vs baseline: 4.3702x; 1.4058x over previous
import functools

import jax
import jax.numpy as jnp
from jax import lax
from jax.experimental import pallas as pl
from jax.experimental.pallas import tpu as pltpu
from jax.experimental.pallas import tpu_sc as plsc

B, M, C, H, W = 4, 4096, 96, 384, 384
HW = H * W
NW = 32
MPW = B * M // NW
CB = C // 8


def _sc_gather(kpts_flat, seg_phys):
    mesh = plsc.VectorSubcoreMesh(core_axis_name="c", subcore_axis_name="s")

    @functools.partial(
        pl.kernel,
        mesh=mesh,
        out_type=jax.ShapeDtypeStruct((B * M * C,), jnp.float32),
        scratch_types=[
            pltpu.VMEM((2 * MPW,), jnp.float32),
            [pltpu.VMEM((MPW * C // 4,), jnp.int32) for _ in range(4)],
            pltpu.VMEM((MPW * C,), jnp.float32),
            [pltpu.SemaphoreType.DMA for _ in range(4)],
            pltpu.SemaphoreType.DMA,
        ],
        compiler_params=pltpu.CompilerParams(needs_layout_passes=False),
    )
    def k(kpts_hbm, seg_hbm, out_hbm, kptv, idxrefs, datav, sems, osem):
        wid = lax.axis_index("s") * 2 + lax.axis_index("c")
        row0 = wid * MPW
        b = row0 // M
        mi = wid % (M // MPW)
        pltpu.sync_copy(kpts_hbm.at[pl.ds(row0 * 2, 2 * MPW)], kptv)

        iota = lax.iota(jnp.int32, 16)
        cbase = b * (C * HW)

        NCH = 4
        GPC = MPW // 16 // NCH
        CW = MPW * C // NCH

        def build(idxref, ch):
            def body(g, carry):
                k0 = (ch * GPC + g) * 16
                ix = (k0 + iota) * 2
                xf = plsc.load_gather(kptv, [ix])
                yf = plsc.load_gather(kptv, [ix + 1])
                xi = (xf * float(W)).astype(jnp.int32)
                yi = (yf * float(H)).astype(jnp.int32)
                xi = jnp.minimum(jnp.maximum(xi, 0), H - 1)
                yi = jnp.minimum(jnp.maximum(yi, 0), W - 1)
                pix = (
                    (xi >> 3) * 3072
                    + (yi >> 7) * 1024
                    + (xi & 7) * 128
                    + (yi & 127)
                    + cbase
                )
                mbase = g * 16
                for cb in range(CB):
                    for cr in range(8):
                        idxref[pl.ds(cb * 1024 + cr * 128 + mbase, 16)] = (
                            pix + (cb * 8 + cr) * HW
                        )
                return carry

            lax.fori_loop(0, GPC, body, 0)

        gathers = []
        for ch in range(NCH):
            build(idxrefs[ch], ch)
            gathers.append(
                pltpu.async_copy(
                    seg_hbm.at[idxrefs[ch]],
                    datav.at[pl.ds(ch * CW, CW)],
                    sems[ch],
                )
            )

        for ch in range(NCH):
            gathers[ch].wait()
            copies = [
                pltpu.async_copy(
                    datav.at[pl.ds(ch * CW + cb * 1024, 1024)],
                    out_hbm.at[
                        pl.ds(
                            b * (C * M) + cb * (8 * M) + (mi * NCH + ch) * 1024,
                            1024,
                        )
                    ],
                    osem,
                )
                for cb in range(CB)
            ]
            for cp in copies:
                cp.wait()

    return k(kpts_flat, seg_phys)


def kernel(original_kpts, segment):
    kpts_flat = original_kpts.reshape(-1)
    seg_phys = (
        segment.reshape(B, C, H // 8, 8, W // 128, 128)
        .transpose(0, 1, 2, 4, 3, 5)
        .reshape(-1)
    )
    out = _sc_gather(kpts_flat, seg_phys)
    return (
        out.reshape(B, CB, M // 128, 8, 128)
        .transpose(0, 2, 4, 1, 3)
        .reshape(B, M, C)
    )


# speedup vs baseline: 4.8535x; 1.1106x over previous
import functools

import jax
import jax.numpy as jnp
from jax import lax
from jax.experimental import pallas as pl
from jax.experimental.pallas import tpu as pltpu
from jax.experimental.pallas import tpu_sc as plsc

B, M, C, H, W = 4, 4096, 96, 384, 384
HW = H * W
NW = 32
MPW = B * M // NW
CB = C // 8


def _sc_gather(kpts_flat, seg_phys):
    mesh = plsc.VectorSubcoreMesh(core_axis_name="c", subcore_axis_name="s")

    @functools.partial(
        pl.kernel,
        mesh=mesh,
        out_type=jax.ShapeDtypeStruct((B * M * C,), jnp.float32),
        scratch_types=[
            pltpu.VMEM((2 * MPW,), jnp.float32),
            [pltpu.VMEM((MPW * C // 4,), jnp.int32) for _ in range(4)],
            pltpu.VMEM((MPW * C,), jnp.float32),
            [pltpu.SemaphoreType.DMA for _ in range(4)],
            pltpu.SemaphoreType.DMA,
        ],
        compiler_params=pltpu.CompilerParams(needs_layout_passes=False),
    )
    def k(kpts_hbm, seg_hbm, out_hbm, kptv, idxrefs, datav, sems, osem):
        wid = lax.axis_index("s") * 2 + lax.axis_index("c")
        row0 = wid * MPW
        b = row0 // M
        mi = wid % (M // MPW)
        pltpu.sync_copy(kpts_hbm.at[pl.ds(row0 * 2, 2 * MPW)], kptv)

        iota = lax.iota(jnp.int32, 16)
        cbase = b * (C * HW)

        NCH = 4
        GPC = MPW // 16 // NCH
        CW = MPW * C // NCH

        def build(idxref, ch):
            def body(g, carry):
                k0 = (ch * GPC + g) * 16
                xb = (k0 >> 7) * 256 + (k0 & 127)
                xf = kptv[pl.ds(xb, 16)]
                yf = kptv[pl.ds(xb + 128, 16)]
                xi = (xf * float(W)).astype(jnp.int32)
                yi = (yf * float(H)).astype(jnp.int32)
                xi = jnp.minimum(jnp.maximum(xi, 0), H - 1)
                yi = jnp.minimum(jnp.maximum(yi, 0), W - 1)
                pix = (
                    (xi >> 3) * 3072
                    + (yi >> 7) * 1024
                    + (xi & 7) * 128
                    + (yi & 127)
                    + cbase
                )
                mbase = g * 16
                for cb in range(CB):
                    for cr in range(8):
                        idxref[pl.ds(cb * 1024 + cr * 128 + mbase, 16)] = (
                            pix + (cb * 8 + cr) * HW
                        )
                return carry

            lax.fori_loop(0, GPC, body, 0)

        gathers = []
        for ch in range(NCH):
            build(idxrefs[ch], ch)
            gathers.append(
                pltpu.async_copy(
                    seg_hbm.at[idxrefs[ch]],
                    datav.at[pl.ds(ch * CW, CW)],
                    sems[ch],
                )
            )

        for ch in range(NCH):
            gathers[ch].wait()
            copies = [
                pltpu.async_copy(
                    datav.at[pl.ds(ch * CW + cb * 1024, 1024)],
                    out_hbm.at[
                        pl.ds(
                            b * (C * M) + cb * (8 * M) + (mi * NCH + ch) * 1024,
                            1024,
                        )
                    ],
                    osem,
                )
                for cb in range(CB)
            ]
            for cp in copies:
                cp.wait()

    return k(kpts_flat, seg_phys)


def kernel(original_kpts, segment):
    kpts_flat = (
        original_kpts.reshape(B, M // 128, 128, 2)
        .transpose(0, 1, 3, 2)
        .reshape(-1)
    )
    seg_phys = (
        segment.reshape(B, C, H // 8, 8, W // 128, 128)
        .transpose(0, 1, 2, 4, 3, 5)
        .reshape(-1)
    )
    out = _sc_gather(kpts_flat, seg_phys)
    return (
        out.reshape(B, CB, M // 128, 8, 128)
        .transpose(0, 2, 4, 1, 3)
        .reshape(B, M, C)
    )
